# baseline (speedup 1.0000x reference)
import functools

import jax
import jax.numpy as jnp
from jax import lax
from jax.experimental import pallas as pl
from jax.experimental.pallas import tpu as pltpu
from jax.experimental.pallas import tpu_sc as plsc

_LANES = 16


def _pooled_lookup(S, SL, V, D):
    info = plsc.get_sparse_core_info()
    NC, NS = info.num_cores, info.num_subcores
    NW = NC * NS
    assert S % NW == 0
    seg_per_w = S // NW
    CSEG = 16
    NBUF = 2
    IDXC = CSEG * SL
    assert seg_per_w % (NBUF * CSEG) == 0
    chunks = seg_per_w // CSEG
    n_vreg = D // _LANES
    idx_words = seg_per_w * SL
    assert idx_words % 8 == 0 and IDXC % 8 == 0

    SEG_SECTIONS = (4, 4, 4, 4)
    assert sum(SEG_SECTIONS) == CSEG
    pieces = []
    off = 0
    for ns in SEG_SECTIONS:
        n = ns * SL
        assert n <= 128 and off % 8 == 0
        pieces.append((off, n))
        off += n
    assert off == IDXC

    mesh = plsc.VectorSubcoreMesh(core_axis_name="c", subcore_axis_name="s")

    @functools.partial(
        pl.kernel,
        mesh=mesh,
        out_type=jax.ShapeDtypeStruct((S, D), jnp.float32),
        scratch_types=[
            pltpu.VMEM((idx_words,), jnp.int32),
        ]
        + [pltpu.VMEM((IDXC, D), jnp.float32) for _ in range(NBUF)]
        + [pltpu.VMEM((CSEG, D), jnp.float32) for _ in range(NBUF)]
        + [pltpu.SemaphoreType.DMA for _ in range(2 * NBUF)],
    )
    def k(idx_hbm, table_hbm, out_hbm, idx_v, *bufs):
        rows = bufs[0:NBUF]
        outs = bufs[NBUF:2 * NBUF]
        sems = bufs[2 * NBUF:3 * NBUF]
        sems_o = bufs[3 * NBUF:4 * NBUF]
        cid = lax.axis_index("c")
        sid = lax.axis_index("s")
        wid = sid * NC + cid
        seg_base = wid * seg_per_w
        pltpu.sync_copy(idx_hbm.at[pl.ds(seg_base * SL, idx_words)], idx_v)

        def fire_piece(g, rows, sem, p):
            o, n = pieces[p]
            pltpu.async_copy(
                table_hbm.at[idx_v.at[pl.ds(g * IDXC + o, n)]],
                rows.at[pl.ds(o, n)],
                sem,
            )

        def fire(g, rows, sem):
            for p in range(len(pieces)):
                fire_piece(g, rows, sem, p)

        def drain_rows(rows, sem):
            pltpu.make_async_copy(
                table_hbm.at[pl.ds(0, IDXC)], rows, sem).wait()

        def drain_out(outb, sem):
            pltpu.make_async_copy(
                outb, out_hbm.at[pl.ds(0, CSEG)], sem).wait()

        def reduce_section(rows, outb, s_lo, s_hi):
            def col_body(v, carry):
                voff = v * _LANES
                for s in range(s_lo, s_hi):
                    acc = rows[s * SL, pl.ds(voff, _LANES)]
                    for j in range(1, SL):
                        acc = acc + rows[s * SL + j, pl.ds(voff, _LANES)]
                    outb[s, pl.ds(voff, _LANES)] = acc
                return carry

            lax.fori_loop(0, n_vreg, col_body, 0)

        def stage(i, g, rowsb, sem, outb, sem_o):
            drain_rows(rowsb, sem)

            @pl.when(i > 0)
            def _():
                drain_out(outb, sem_o)

            s_lo = 0
            for p, ns in enumerate(SEG_SECTIONS):
                reduce_section(rowsb, outb, s_lo, s_lo + ns)
                s_lo += ns

                @pl.when(g + NBUF < chunks)
                def _():
                    fire_piece(g + NBUF, rowsb, sem, p)

            pltpu.async_copy(
                outb, out_hbm.at[pl.ds(seg_base + g * CSEG, CSEG)], sem_o)

        def body(i, carry):
            for b in range(NBUF):
                stage(i, NBUF * i + b, rows[b], sems[b], outs[b], sems_o[b])
            return carry

        for b in range(NBUF):
            fire(b, rows[b], sems[b])
        lax.fori_loop(0, chunks // NBUF, body, 0)
        for b in range(NBUF):
            drain_out(outs[b], sems_o[b])

    return k


def kernel(sentences, table):
    B, T, SL = sentences.shape
    V, D = table.shape
    S = B * T
    idx_flat = sentences.reshape(S * SL).astype(jnp.int32)
    k = _pooled_lookup(S, SL, V, D)
    out_flat = k(idx_flat, table)
    return out_flat.reshape(B, T, D)


# speedup vs baseline: 1.0625x; 1.0625x over previous
import functools

import jax
import jax.numpy as jnp
from jax import lax
from jax.experimental import pallas as pl
from jax.experimental.pallas import tpu as pltpu
from jax.experimental.pallas import tpu_sc as plsc

_LANES = 16


def _pooled_lookup(B, T, SL, V, D):
    S = B * T
    info = plsc.get_sparse_core_info()
    NC, NS = info.num_cores, info.num_subcores
    NW = NC * NS
    assert B % NW == 0 and T % 2 == 0
    seg_per_w = S // NW
    bat_per_w = B // NW
    CSEG = T // 2
    IDXC = CSEG * SL
    chunks = seg_per_w // CSEG
    assert chunks % 2 == 0
    n_vreg = D // _LANES
    idx_words = seg_per_w * SL
    assert idx_words % 8 == 0

    PAD = (IDXC % 8)
    pieces_ev = ((0, 120), (120, 120), (240, 24))
    pieces_od = ((0, 128), (128, 128), (256, 8))
    rows_len = IDXC + PAD
    fire_after_ev = (6, 12, 13)
    fire_after_od = (7, 13, 13)

    mesh = plsc.VectorSubcoreMesh(core_axis_name="c", subcore_axis_name="s")

    @functools.partial(
        pl.kernel,
        mesh=mesh,
        out_type=jax.ShapeDtypeStruct((B, T, D), jnp.float32),
        scratch_types=[
            pltpu.VMEM((idx_words,), jnp.int32),
            pltpu.VMEM((rows_len, D), jnp.float32),
            pltpu.VMEM((rows_len, D), jnp.float32),
            pltpu.VMEM((T, D), jnp.float32),
            pltpu.SemaphoreType.DMA,
            pltpu.SemaphoreType.DMA,
            pltpu.SemaphoreType.DMA,
        ],
    )
    def k(idx_hbm, table_hbm, out_hbm, idx_v, rows_a, rows_b, out_f,
          sem_a, sem_b, sem_o):
        cid = lax.axis_index("c")
        sid = lax.axis_index("s")
        wid = sid * NC + cid
        bat_base = wid * bat_per_w
        pltpu.sync_copy(
            idx_hbm.at[pl.ds(wid * idx_words, idx_words)], idx_v)

        def fire_piece(ii, rowsb, sem, p, odd):
            o, n = (pieces_od if odd else pieces_ev)[p]
            base = ii * (2 * IDXC) + ((IDXC - PAD) if odd else 0)
            pltpu.async_copy(
                table_hbm.at[idx_v.at[pl.ds(base + o, n)]],
                rowsb.at[pl.ds(o, n)],
                sem,
            )

        def fire(ii, rowsb, sem, odd):
            for p in range(3):
                fire_piece(ii, rowsb, sem, p, odd)

        def drain_rows(rowsb, sem, odd):
            n = rows_len if odd else IDXC + PAD
            pltpu.make_async_copy(
                table_hbm.at[pl.ds(0, n)], rowsb.at[pl.ds(0, n)], sem).wait()

        def drain_out():
            pltpu.make_async_copy(out_f, out_hbm.at[0], sem_o).wait()

        def reduce_section(rowsb, s_lo, s_hi, shift, odd):
            def col_body(v, carry):
                voff = v * _LANES
                for s in range(s_lo, s_hi):
                    acc = rowsb[shift + s * SL, pl.ds(voff, _LANES)]
                    for j in range(1, SL):
                        acc = acc + rowsb[
                            shift + s * SL + j, pl.ds(voff, _LANES)]
                    out_f[(CSEG if odd else 0) + s, pl.ds(voff, _LANES)] = acc
                return carry

            lax.fori_loop(0, n_vreg, col_body, 0)

        def stage(i, rowsb, sem, odd):
            drain_rows(rowsb, sem, odd)

            if not odd:
                @pl.when(i > 0)
                def _():
                    drain_out()

            shift = PAD if odd else 0
            fire_after = fire_after_od if odd else fire_after_ev
            s_lo = 0
            for p in range(3):
                s_hi = fire_after[p]
                if s_hi > s_lo:
                    reduce_section(rowsb, s_lo, s_hi, shift, odd)
                    s_lo = s_hi

                @pl.when(i + 1 < chunks // 2)
                def _():
                    fire_piece(i + 1, rowsb, sem, p, odd)

        def body(i, carry):
            stage(i, rows_a, sem_a, False)
            stage(i, rows_b, sem_b, True)
            pltpu.async_copy(out_f, out_hbm.at[bat_base + i], sem_o)
            return carry

        fire(0, rows_a, sem_a, False)
        fire(0, rows_b, sem_b, True)
        lax.fori_loop(0, chunks // 2, body, 0)
        drain_out()

    return k


def kernel(sentences, table):
    B, T, SL = sentences.shape
    V, D = table.shape
    idx_flat = sentences.reshape(B * T * SL).astype(jnp.int32)
    k = _pooled_lookup(B, T, SL, V, D)
    return k(idx_flat, table)
